# Initial kernel scaffold; baseline (speedup 1.0000x reference)
import jax
import jax.numpy as jnp
from jax.experimental import pallas as pl


def kernel(x, edge_index, e, W_msg, w1, b1, w2, b2, v1, vb1, v2, vb2):
    raise NotImplementedError("write your pallas kernel here")


# speedup vs baseline: 3.6249x; 3.6249x over previous
import functools

import jax
import jax.numpy as jnp
from jax import lax
from jax.experimental import pallas as pl
from jax.experimental.pallas import tpu as pltpu
from jax.experimental.pallas import tpu_sc as plsc

N = 10000
E = 320000
D_NODE = 128
D_EDGE = 16
HIDDEN = 64

NC = 2
NS = 16
W = 128

_f32 = jnp.float32


def _sds(shape, dtype=_f32):
    return jax.ShapeDtypeStruct(shape, dtype)


def _tc_pre(x, W_msg, w1):
    def k(x_ref, wm_ref, w1_ref, xa_ref, xb_ref):
        w1a = w1_ref[0:D_NODE, :]
        w1b = w1_ref[D_NODE:2 * D_NODE, :]
        w1c = w1_ref[2 * D_NODE:2 * D_NODE + D_EDGE, :]
        A = w1a + jnp.dot(wm_ref[...], w1c, preferred_element_type=_f32)
        xa_ref[...] = jnp.dot(x_ref[...], A, preferred_element_type=_f32)
        xb_ref[...] = jnp.dot(x_ref[...], w1b, preferred_element_type=_f32)

    return pl.pallas_call(
        k,
        out_shape=(_sds((N, HIDDEN)), _sds((N, HIDDEN))),
    )(x, W_msg, w1)


def _sc_gather(xa, xb, src2d, dst2d):
    mesh = plsc.VectorSubcoreMesh(core_axis_name="c", subcore_axis_name="s")

    @functools.partial(
        pl.kernel,
        mesh=mesh,
        out_type=_sds((E, HIDDEN)),
        scratch_types=[
            pltpu.VMEM((W, HIDDEN), _f32),
            pltpu.SemaphoreType.DMA,
        ],
        compiler_params=pltpu.CompilerParams(use_tc_tiling_on_sc=False),
    )
    def k(xa_hbm, xb_hbm, src_hbm, dst_hbm, g_hbm, bufa, sem):
        def body(isrc_v, idst_v, o_v):
            ca = pltpu.async_copy(xa_hbm.at[isrc_v.at[0]], bufa, sem)
            pltpu.sync_copy(xb_hbm.at[idst_v.at[0]], o_v)
            ca.wait()

            @pl.loop(0, W)
            def _(r):
                for c in range(0, HIDDEN, 16):
                    slc = (pl.ds(r, 1), pl.ds(c, 16))
                    o_v.at[slc][...] = o_v.at[slc][...] + bufa.at[slc][...]

        pltpu.emit_pipeline(
            body,
            grid=(E // W,),
            in_specs=[
                pl.BlockSpec((1, W), lambda i: (0, i)),
                pl.BlockSpec((1, W), lambda i: (0, i)),
            ],
            out_specs=[pl.BlockSpec((W, HIDDEN), lambda i: (i, 0))],
            core_axis_name=("c", "s"),
            dimension_semantics=(pltpu.PARALLEL,),
        )(src_hbm, dst_hbm, g_hbm)

    return k(xa, xb, src2d, dst2d)


def _tc_edge(g, e, w1c, b1r, w2, b2r):
    BE = 16000

    def k(g_ref, e_ref, wc_ref, b1_ref, w2_ref, b2_ref, o_ref):
        pre = g_ref[...] + jnp.dot(e_ref[...], wc_ref[...],
                                   preferred_element_type=_f32) + b1_ref[...]
        h = jnp.maximum(pre, 0.0)
        o_ref[...] = e_ref[...] + jnp.dot(h, w2_ref[...],
                                          preferred_element_type=_f32) + b2_ref[...]

    return pl.pallas_call(
        k,
        grid=(E // BE,),
        in_specs=[
            pl.BlockSpec((BE, HIDDEN), lambda i: (i, 0)),
            pl.BlockSpec((BE, D_EDGE), lambda i: (i, 0)),
            pl.BlockSpec((D_EDGE, HIDDEN), lambda i: (0, 0)),
            pl.BlockSpec((1, HIDDEN), lambda i: (0, 0)),
            pl.BlockSpec((HIDDEN, D_EDGE), lambda i: (0, 0)),
            pl.BlockSpec((1, D_EDGE), lambda i: (0, 0)),
        ],
        out_specs=pl.BlockSpec((BE, D_EDGE), lambda i: (i, 0)),
        out_shape=_sds((E, D_EDGE)),
    )(g, e, w1c, b1r, w2, b2r)


def _sc_scatter(e_new, dst2d, zeros_nk, ones_wk):
    mesh = plsc.VectorSubcoreMesh(core_axis_name="c", subcore_axis_name="s")
    ROWS = N // NS

    @functools.partial(
        pl.kernel,
        mesh=mesh,
        out_type=(_sds((NC * N, D_EDGE)), _sds((NC * N, D_EDGE))),
        scratch_types=[
            pltpu.VMEM((W, D_EDGE), _f32),
            pltpu.VMEM_SHARED((N, D_EDGE), _f32),
            pltpu.VMEM_SHARED((N, D_EDGE), _f32),
        ],
        compiler_params=pltpu.CompilerParams(use_tc_tiling_on_sc=False),
    )
    def k(enew_hbm, dst_hbm, z_hbm, ones_hbm, sums_hbm, cnt_hbm,
          ones_v, sums_sh, cnt_sh):
        cid = lax.axis_index("c")
        sid = lax.axis_index("s")
        r0 = sid * ROWS
        pltpu.sync_copy(z_hbm.at[pl.ds(r0, ROWS)], sums_sh.at[pl.ds(r0, ROWS)])
        pltpu.sync_copy(z_hbm.at[pl.ds(r0, ROWS)], cnt_sh.at[pl.ds(r0, ROWS)])
        pltpu.sync_copy(ones_hbm, ones_v)
        plsc.subcore_barrier()

        def body(e_v, i_v):
            pltpu.sync_copy(e_v, sums_sh.at[i_v.at[0]], add=True)
            pltpu.sync_copy(ones_v, cnt_sh.at[i_v.at[0]], add=True)

        pltpu.emit_pipeline(
            body,
            grid=(E // W,),
            in_specs=[
                pl.BlockSpec((W, D_EDGE), lambda i: (i, 0)),
                pl.BlockSpec((1, W), lambda i: (0, i)),
            ],
            out_specs=[],
            core_axis_name=("c", "s"),
            dimension_semantics=(pltpu.PARALLEL,),
        )(enew_hbm, dst_hbm)

        plsc.subcore_barrier()
        off = cid * N + r0
        pltpu.sync_copy(sums_sh.at[pl.ds(r0, ROWS)], sums_hbm.at[pl.ds(off, ROWS)])
        pltpu.sync_copy(cnt_sh.at[pl.ds(r0, ROWS)], cnt_hbm.at[pl.ds(off, ROWS)])

    return k(e_new, dst2d, zeros_nk, ones_wk)


def _tc_node(x, sums_p, cnt_p, v1a, v1b, vb1r, v2, vb2r):
    def k(x_ref, s_ref, c_ref, v1a_ref, v1b_ref, vb1_ref, v2_ref, vb2_ref,
          o_ref):
        s = s_ref[0:N, :] + s_ref[N:2 * N, :]
        cnt = c_ref[0:N, 0:1] + c_ref[N:2 * N, 0:1]
        m = s / jnp.maximum(cnt, 1.0)
        pre = (jnp.dot(x_ref[...], v1a_ref[...], preferred_element_type=_f32)
               + jnp.dot(m, v1b_ref[...], preferred_element_type=_f32)
               + vb1_ref[...])
        hv = jnp.maximum(pre, 0.0)
        o_ref[...] = x_ref[...] + jnp.dot(hv, v2_ref[...],
                                          preferred_element_type=_f32) + vb2_ref[...]

    return pl.pallas_call(
        k,
        out_shape=_sds((N, D_NODE)),
    )(x, sums_p, cnt_p, v1a, v1b, vb1r, v2, vb2r)


def kernel(x, edge_index, e, W_msg, w1, b1, w2, b2, v1, vb1, v2, vb2):
    src2d = edge_index[0:1, :].astype(jnp.int32)
    dst2d = edge_index[1:2, :].astype(jnp.int32)

    xa, xb = _tc_pre(x, W_msg, w1)
    g = _sc_gather(xa, xb, src2d, dst2d)

    w1c = w1[2 * D_NODE:2 * D_NODE + D_EDGE, :]
    e_new = _tc_edge(g, e, w1c, b1.reshape(1, HIDDEN), w2,
                     b2.reshape(1, D_EDGE))

    zeros_nk = jnp.zeros((N, D_EDGE), _f32)
    ones_wk = jnp.ones((W, D_EDGE), _f32)
    sums_p, cnt_p = _sc_scatter(e_new, dst2d, zeros_nk, ones_wk)

    x_new = _tc_node(x, sums_p, cnt_p, v1[0:D_NODE, :],
                     v1[D_NODE:D_NODE + D_EDGE, :],
                     vb1.reshape(1, HIDDEN), v2, vb2.reshape(1, D_NODE))
    return (x_new, e_new)
